# baseline (speedup 1.0000x reference)
import jax
import jax.numpy as jnp
from jax import lax
from jax.experimental import pallas as pl
from jax.experimental.pallas import tpu as pltpu


_MIX = 0x45D9F3B
_NB = 999999

_N = 16384 * 200
_CH = _N // 16
_C = _N // _CH
_D = 4


def _bucket(v):
    u = v.astype(jnp.uint32)
    h = u ^ (u >> 16)
    h = h * jnp.uint32(_MIX)
    h = h ^ (h >> 16)
    h = h * jnp.uint32(_MIX)
    h = h ^ (h >> 16)
    t = (h % jnp.uint32(_NB)).astype(jnp.int32)
    return jnp.where(v == 0, 0, t + 1)


def _body(x_hbm, o_hbm, ibuf, obuf, isem, osem):
    def in_copy(i, slot):
        return pltpu.make_async_copy(
            x_hbm.at[pl.ds(i * _CH, _CH)], ibuf.at[slot], isem.at[slot])

    def out_copy(i, slot):
        return pltpu.make_async_copy(
            obuf.at[slot], o_hbm.at[pl.ds(i * _CH, _CH)], osem.at[slot])

    for i in range(_D):
        in_copy(i, i).start()
    for i in range(_C):
        slot = i % _D
        in_copy(i, slot).wait()
        if i >= _D:
            out_copy(i - _D, slot).wait()
        obuf[slot] = _bucket(ibuf[slot])
        out_copy(i, slot).start()
        if i + _D < _C:
            in_copy(i + _D, slot).start()
    for i in range(_C - _D, _C):
        out_copy(i, i % _D).wait()


def kernel(x):
    xf = x.reshape(_N)
    out = pl.pallas_call(
        _body,
        out_shape=jax.ShapeDtypeStruct((_N,), jnp.int32),
        in_specs=[pl.BlockSpec(memory_space=pl.ANY)],
        out_specs=pl.BlockSpec(memory_space=pl.ANY),
        scratch_shapes=[
            pltpu.VMEM((_D, _CH), jnp.int32),
            pltpu.VMEM((_D, _CH), jnp.int32),
            pltpu.SemaphoreType.DMA((_D,)),
            pltpu.SemaphoreType.DMA((_D,)),
        ],
    )(xf)
    return out.reshape(16384, 200)


# speedup vs baseline: 7.2396x; 7.2396x over previous
import jax
import jax.numpy as jnp
from jax import lax
from jax.experimental import pallas as pl
from jax.experimental.pallas import tpu as pltpu


_MIX = 0x45D9F3B
_NB = 999999


def _bucket(v):
    u = v.astype(jnp.uint32)
    h = u ^ (u >> 16)
    h = h * jnp.uint32(_MIX)
    h = h ^ (h >> 16)
    h = h * jnp.uint32(_MIX)
    h = h ^ (h >> 16)
    t = (h % jnp.uint32(_NB)).astype(jnp.int32)
    return jnp.where(v == 0, 0, t + 1)


def _tc_body(x_ref, o_ref):
    o_ref[...] = _bucket(x_ref[...])


def kernel(x):
    xt = x.T
    out_t = pl.pallas_call(
        _tc_body,
        out_shape=jax.ShapeDtypeStruct((200, 16384), jnp.int32),
        grid=(8,),
        in_specs=[pl.BlockSpec((200, 2048), lambda i: (0, i))],
        out_specs=pl.BlockSpec((200, 2048), lambda i: (0, i)),
    )(xt)
    return out_t.T


# speedup vs baseline: 9.1836x; 1.2685x over previous
import jax
import jax.numpy as jnp
from jax import lax
from jax.experimental import pallas as pl
from jax.experimental.pallas import tpu as pltpu


_MIX = 0x45D9F3B
_NB = 999999

_ROWS = 200
_COLS = 16384
_R = 8
_C = _ROWS // _R
_D = 5


def _bucket(v):
    u = v.astype(jnp.uint32)
    h = u ^ (u >> 16)
    h = h * jnp.uint32(_MIX)
    h = h ^ (h >> 16)
    h = h * jnp.uint32(_MIX)
    h = h ^ (h >> 16)
    t = (h % jnp.uint32(_NB)).astype(jnp.int32)
    return jnp.where(v == 0, 0, t + 1)


def _body(x_hbm, o_hbm, ibuf, obuf, isem, osem):
    def in_copy(i, slot):
        return pltpu.make_async_copy(
            x_hbm.at[pl.ds(i * _R, _R)], ibuf.at[slot], isem.at[slot])

    def out_copy(i, slot):
        return pltpu.make_async_copy(
            obuf.at[slot], o_hbm.at[pl.ds(i * _R, _R)], osem.at[slot])

    for i in range(_D):
        in_copy(i, i).start()
    for i in range(_C):
        slot = i % _D
        in_copy(i, slot).wait()
        if i >= _D:
            out_copy(i - _D, slot).wait()
        obuf[slot] = _bucket(ibuf[slot])
        out_copy(i, slot).start()
        if i + _D < _C:
            in_copy(i + _D, slot).start()
    for i in range(_C - _D, _C):
        out_copy(i, i % _D).wait()


def kernel(x):
    xt = x.T
    out_t = pl.pallas_call(
        _body,
        out_shape=jax.ShapeDtypeStruct((_ROWS, _COLS), jnp.int32),
        in_specs=[pl.BlockSpec(memory_space=pltpu.MemorySpace.HBM)],
        out_specs=pl.BlockSpec(memory_space=pltpu.MemorySpace.HBM),
        scratch_shapes=[
            pltpu.VMEM((_D, _R, _COLS), jnp.int32),
            pltpu.VMEM((_D, _R, _COLS), jnp.int32),
            pltpu.SemaphoreType.DMA((_D,)),
            pltpu.SemaphoreType.DMA((_D,)),
        ],
    )(xt)
    return out_t.T
